# Initial kernel scaffold; baseline (speedup 1.0000x reference)
import jax
import jax.numpy as jnp
from jax.experimental import pallas as pl


def kernel(seq_len, relative_position_matrix):
    raise NotImplementedError("write your pallas kernel here")


# speedup vs baseline: 6.0259x; 6.0259x over previous
import functools

import jax
import jax.numpy as jnp
from jax import lax
from jax.experimental import pallas as pl
from jax.experimental.pallas import tpu as pltpu
from jax.experimental.pallas import tpu_sc as plsc

_SEQ = 1024
_D = 64
_NW = 32
_ROWS_PER_W = _SEQ // _NW
_SLAB = _SEQ + _ROWS_PER_W
_ROW_W = _SEQ * _D


def _sc_toeplitz(ftable_flat):
    mesh = plsc.VectorSubcoreMesh(core_axis_name="c", subcore_axis_name="s",
                                  num_cores=2)

    @functools.partial(
        pl.kernel,
        mesh=mesh,
        out_type=jax.ShapeDtypeStruct((_SEQ * _SEQ * _D,), jnp.float32),
        scratch_types=[
            pltpu.VMEM((_SLAB * _D,), jnp.float32),
            pltpu.SemaphoreType.DMA,
        ],
    )
    def k(ftable_hbm, out_hbm, slab, sem):
        wid = lax.axis_index("s") * 2 + lax.axis_index("c")
        i0 = wid * _ROWS_PER_W
        start = ((_SEQ * 2 - _ROWS_PER_W) - i0) * _D
        pltpu.sync_copy(ftable_hbm.at[pl.ds(start, _SLAB * _D)], slab)
        copies = []
        for r in range(_ROWS_PER_W):
            copies.append(
                pltpu.async_copy(
                    slab.at[pl.ds((_ROWS_PER_W - 1 - r) * _D, _ROW_W)],
                    out_hbm.at[pl.ds((i0 + r) * _ROW_W, _ROW_W)],
                    sem,
                )
            )
        for c in copies:
            c.wait()

    return k(ftable_flat)


def kernel(seq_len, relative_position_matrix):
    del seq_len
    ftable = jnp.flip(relative_position_matrix, axis=0).reshape(-1)
    out_flat = _sc_toeplitz(ftable)
    return out_flat.reshape(_SEQ, _SEQ, _D)


# speedup vs baseline: 6.5432x; 1.0858x over previous
import functools

import jax
import jax.numpy as jnp
from jax import lax
from jax.experimental import pallas as pl
from jax.experimental.pallas import tpu as pltpu
from jax.experimental.pallas import tpu_sc as plsc

_SEQ = 1024
_D = 64
_TBL = 2 * _SEQ
_ROWS_PER_W = _SEQ // 32


def _sc_toeplitz(fu):
    mesh = plsc.VectorSubcoreMesh(core_axis_name="c", subcore_axis_name="s",
                                  num_cores=2)

    @functools.partial(
        pl.kernel,
        mesh=mesh,
        out_type=jax.ShapeDtypeStruct((_SEQ, _SEQ, _D), jnp.float32),
        scratch_types=[
            pltpu.VMEM_SHARED((_TBL, _D), jnp.float32),
            pltpu.SemaphoreType.DMA,
        ],
        compiler_params=pltpu.CompilerParams(use_tc_tiling_on_sc=True),
    )
    def k(fu_hbm, out_hbm, spmem, sem):
        c = lax.axis_index("c")
        s = lax.axis_index("s")
        chunk = _TBL // 16
        pltpu.sync_copy(fu_hbm.at[pl.ds(s * chunk, chunk)],
                        spmem.at[pl.ds(s * chunk, chunk)])
        plsc.subcore_barrier()
        wid = c * 16 + s
        i0 = wid * _ROWS_PER_W
        copies = []
        for r in range(_ROWS_PER_W):
            i = i0 + r
            copies.append(
                pltpu.async_copy(
                    spmem.at[pl.ds(_SEQ - 1 - i, _SEQ)],
                    out_hbm.at[i],
                    sem,
                )
            )
        for cp in copies:
            cp.wait()

    return k(fu)


def kernel(seq_len, relative_position_matrix):
    del seq_len
    fu = jnp.flip(relative_position_matrix, axis=0)[_SEQ:3 * _SEQ, :]
    return _sc_toeplitz(fu)
